# baseline (speedup 1.0000x reference)
import functools

import jax
import jax.numpy as jnp
from jax import lax
from jax.experimental import pallas as pl
from jax.experimental.pallas import tpu as pltpu
from jax.experimental.pallas import tpu_sc as plsc

B = 16384
F = 26
D = 32
BF = B * F
GW = 128
NROWS = BF // GW
NW = 32
ROWS_PER_W = NROWS // NW
CHUNK_ROWS = 8
NCHUNK = ROWS_PER_W // CHUNK_ROWS
CHUNK_IDX = CHUNK_ROWS * GW
LANES = 16
OFF_PERIOD = 13


def _emb_lookup(ids2d, offs13, table):
    mesh = plsc.VectorSubcoreMesh(core_axis_name="c", subcore_axis_name="s")

    @functools.partial(
        pl.kernel,
        mesh=mesh,
        out_type=jax.ShapeDtypeStruct((BF, D), jnp.float32),
        scratch_types=[
            pltpu.VMEM((ROWS_PER_W, GW), jnp.int32),
            pltpu.VMEM((OFF_PERIOD, GW), jnp.int32),
            pltpu.VMEM((CHUNK_IDX, D), jnp.float32),
            pltpu.SemaphoreType.DMA,
        ],
        compiler_params=pltpu.CompilerParams(use_tc_tiling_on_sc=False),
    )
    def k(ids_hbm, off_hbm, table_hbm, out_hbm, idx_v, off_v, buf, sem):
        cid = lax.axis_index("c")
        sid = lax.axis_index("s")
        wid = sid * 2 + cid
        row0 = wid * ROWS_PER_W

        pltpu.sync_copy(off_hbm, off_v)
        pltpu.sync_copy(ids_hbm.at[pl.ds(row0 * 1, ROWS_PER_W)], idx_v)

        def add_row(r, carry):
            phase = lax.rem(row0 + r, OFF_PERIOD)
            for c in range(GW // LANES):
                off = off_v[phase, pl.ds(c * LANES, LANES)]
                cur = idx_v[r, pl.ds(c * LANES, LANES)]
                idx_v[r, pl.ds(c * LANES, LANES)] = cur + off
            return carry

        lax.fori_loop(0, ROWS_PER_W, add_row, 0)

        def chunk(t, carry):
            copies = []
            for j in range(CHUNK_ROWS):
                r = t * CHUNK_ROWS + j
                cp = pltpu.async_copy(
                    table_hbm.at[idx_v.at[r]],
                    buf.at[pl.ds(j * GW, GW)],
                    sem,
                )
                copies.append(cp)
            for cp in copies:
                cp.wait()
            pltpu.sync_copy(
                buf, out_hbm.at[pl.ds(row0 * GW + t * CHUNK_IDX, CHUNK_IDX)]
            )
            return carry

        lax.fori_loop(0, NCHUNK, chunk, 0)

    return k(ids2d, offs13, table)


def kernel(input_ids, offsets, table):
    ids2d = input_ids.reshape(NROWS, GW)
    offs13 = jnp.tile(offsets, OFF_PERIOD * GW // F).reshape(OFF_PERIOD, GW)
    out = _emb_lookup(ids2d, offs13, table)
    return out.reshape(B, F, D)
